# baseline (device time: 55222 ns/iter reference)
import os

import jax
import jax.numpy as jnp
from jax import lax
from jax.experimental import pallas as pl
from jax.experimental.pallas import tpu as pltpu

N_DEV = 4

_KMODE = os.environ.get("KMODE", "full")


def kernel(x, w_mat):
    m_per, k = x.shape
    _, n_per = w_mat.shape
    m_total = N_DEV * m_per
    m_half = m_per // 2
    m_q = m_per // 4
    do_gemm = _KMODE in ("full", "gemm")
    do_comm = _KMODE in ("full", "comm")
    probe = _KMODE if _KMODE in ("bar", "send1", "p0") else None

    if probe is not None:
        def probe_body(x_ref, w_ref, out_ref, gath_ref, send_sems, recv_sems):
            my = lax.axis_index("i")
            left = lax.rem(my - 1 + N_DEV, N_DEV)
            right = lax.rem(my + 1, N_DEV)
            my_row = my * m_per
            left_row = left * m_per
            right_row = right * m_per

            barrier_sem = pltpu.get_barrier_semaphore()
            for nbr in [left, right]:
                pl.semaphore_signal(
                    barrier_sem, inc=1,
                    device_id=(nbr,), device_id_type=pl.DeviceIdType.MESH,
                )
            pl.semaphore_wait(barrier_sem, 2)
            if probe == "bar":
                return
            gath_ref[pl.ds(my_row, m_per), :] = x_ref[...]
            s_r0 = pltpu.make_async_remote_copy(
                src_ref=gath_ref.at[pl.ds(my_row, m_per), :],
                dst_ref=gath_ref.at[pl.ds(my_row, m_per), :],
                send_sem=send_sems.at[0], recv_sem=recv_sems.at[0],
                device_id=(right,), device_id_type=pl.DeviceIdType.MESH,
            )
            s_r0.start()
            if probe == "p0":
                s_l0 = pltpu.make_async_remote_copy(
                    src_ref=gath_ref.at[pl.ds(my_row, m_per), :],
                    dst_ref=gath_ref.at[pl.ds(my_row, m_per), :],
                    send_sem=send_sems.at[1], recv_sem=recv_sems.at[1],
                    device_id=(left,), device_id_type=pl.DeviceIdType.MESH,
                )
                s_l0.start()
            r_l0 = pltpu.make_async_remote_copy(
                src_ref=gath_ref.at[pl.ds(left_row, m_per), :],
                dst_ref=gath_ref.at[pl.ds(left_row, m_per), :],
                send_sem=send_sems.at[0], recv_sem=recv_sems.at[0],
                device_id=(left,), device_id_type=pl.DeviceIdType.MESH,
            )
            r_l0.wait_recv()
            if probe == "p0":
                r_r0 = pltpu.make_async_remote_copy(
                    src_ref=gath_ref.at[pl.ds(right_row, m_per), :],
                    dst_ref=gath_ref.at[pl.ds(right_row, m_per), :],
                    send_sem=send_sems.at[1], recv_sem=recv_sems.at[1],
                    device_id=(right,), device_id_type=pl.DeviceIdType.MESH,
                )
                r_r0.wait_recv()
                s_l0.wait_send()
            s_r0.wait_send()

        return pl.pallas_call(
            probe_body,
            out_shape=jax.ShapeDtypeStruct((m_total, n_per), jnp.float32),
            in_specs=[pl.BlockSpec(memory_space=pltpu.VMEM),
                      pl.BlockSpec(memory_space=pltpu.VMEM)],
            out_specs=pl.BlockSpec(memory_space=pltpu.VMEM),
            scratch_shapes=[
                pltpu.VMEM((m_total, k), jnp.float32),
                pltpu.SemaphoreType.DMA((2,)),
                pltpu.SemaphoreType.DMA((2,)),
            ],
            compiler_params=pltpu.CompilerParams(collective_id=0),
        )(x, w_mat)

    if _KMODE == "gemm":
        def gemm_body(x_ref, w_ref, out_ref):
            for c in range(N_DEV):
                out_ref[pl.ds(c * m_per, m_per), :] = jnp.dot(
                    x_ref[...], w_ref[...],
                    preferred_element_type=jnp.float32,
                )
        return pl.pallas_call(
            gemm_body,
            out_shape=jax.ShapeDtypeStruct((m_total, n_per), jnp.float32),
            in_specs=[pl.BlockSpec(memory_space=pltpu.VMEM),
                      pl.BlockSpec(memory_space=pltpu.VMEM)],
            out_specs=pl.BlockSpec(memory_space=pltpu.VMEM),
        )(x, w_mat)

    def body(x_ref, w_ref, out_ref, gath_ref, send_sems, recv_sems):
        my = lax.axis_index("i")
        left = lax.rem(my - 1 + N_DEV, N_DEV)
        right = lax.rem(my + 1, N_DEV)
        opp = lax.rem(my + 2, N_DEV)

        my_row = my * m_per
        left_row = left * m_per
        right_row = right * m_per
        opp_row = opp * m_per

        barrier_sem = pltpu.get_barrier_semaphore()
        for nbr in [left, right]:
            pl.semaphore_signal(
                barrier_sem, inc=1,
                device_id=(nbr,), device_id_type=pl.DeviceIdType.MESH,
            )
        pl.semaphore_wait(barrier_sem, 2)

        gath_ref[pl.ds(my_row, m_per), :] = x_ref[...]

        s_r0 = pltpu.make_async_remote_copy(
            src_ref=gath_ref.at[pl.ds(my_row, m_per), :],
            dst_ref=gath_ref.at[pl.ds(my_row, m_per), :],
            send_sem=send_sems.at[0], recv_sem=recv_sems.at[0],
            device_id=(right,), device_id_type=pl.DeviceIdType.MESH,
        )
        s_r0.start()
        s_l0 = pltpu.make_async_remote_copy(
            src_ref=gath_ref.at[pl.ds(my_row, m_per), :],
            dst_ref=gath_ref.at[pl.ds(my_row, m_per), :],
            send_sem=send_sems.at[1], recv_sem=recv_sems.at[1],
            device_id=(left,), device_id_type=pl.DeviceIdType.MESH,
        )
        s_l0.start()

        if do_gemm:
            out_ref[pl.ds(my_row, m_per), :] = jnp.dot(
                gath_ref[pl.ds(my_row, m_per), :], w_ref[...],
                preferred_element_type=jnp.float32,
            )

        r_l0 = pltpu.make_async_remote_copy(
            src_ref=gath_ref.at[pl.ds(left_row, m_per), :],
            dst_ref=gath_ref.at[pl.ds(left_row, m_per), :],
            send_sem=send_sems.at[0], recv_sem=recv_sems.at[0],
            device_id=(left,), device_id_type=pl.DeviceIdType.MESH,
        )
        r_l0.wait_recv()
        s_r1a = pltpu.make_async_remote_copy(
            src_ref=gath_ref.at[pl.ds(left_row, m_q), :],
            dst_ref=gath_ref.at[pl.ds(left_row, m_q), :],
            send_sem=send_sems.at[2], recv_sem=recv_sems.at[2],
            device_id=(right,), device_id_type=pl.DeviceIdType.MESH,
        )
        s_r1a.start()
        s_r1b = pltpu.make_async_remote_copy(
            src_ref=gath_ref.at[pl.ds(left_row + m_q, m_q), :],
            dst_ref=gath_ref.at[pl.ds(left_row + m_q, m_q), :],
            send_sem=send_sems.at[3], recv_sem=recv_sems.at[3],
            device_id=(right,), device_id_type=pl.DeviceIdType.MESH,
        )
        s_r1b.start()

        r_r0 = pltpu.make_async_remote_copy(
            src_ref=gath_ref.at[pl.ds(right_row, m_per), :],
            dst_ref=gath_ref.at[pl.ds(right_row, m_per), :],
            send_sem=send_sems.at[1], recv_sem=recv_sems.at[1],
            device_id=(right,), device_id_type=pl.DeviceIdType.MESH,
        )
        r_r0.wait_recv()
        s_l1a = pltpu.make_async_remote_copy(
            src_ref=gath_ref.at[pl.ds(right_row + m_half, m_q), :],
            dst_ref=gath_ref.at[pl.ds(right_row + m_half, m_q), :],
            send_sem=send_sems.at[4], recv_sem=recv_sems.at[4],
            device_id=(left,), device_id_type=pl.DeviceIdType.MESH,
        )
        s_l1a.start()
        s_l1b = pltpu.make_async_remote_copy(
            src_ref=gath_ref.at[pl.ds(right_row + m_half + m_q, m_q), :],
            dst_ref=gath_ref.at[pl.ds(right_row + m_half + m_q, m_q), :],
            send_sem=send_sems.at[5], recv_sem=recv_sems.at[5],
            device_id=(left,), device_id_type=pl.DeviceIdType.MESH,
        )
        s_l1b.start()

        if do_gemm:
            out_ref[pl.ds(left_row, m_per), :] = jnp.dot(
                gath_ref[pl.ds(left_row, m_per), :], w_ref[...],
                preferred_element_type=jnp.float32,
            )
            out_ref[pl.ds(right_row, m_per), :] = jnp.dot(
                gath_ref[pl.ds(right_row, m_per), :], w_ref[...],
                preferred_element_type=jnp.float32,
            )

        def opp_recv(strip, sem_idx, nbr):
            return pltpu.make_async_remote_copy(
                src_ref=gath_ref.at[pl.ds(opp_row + strip * m_q, m_q), :],
                dst_ref=gath_ref.at[pl.ds(opp_row + strip * m_q, m_q), :],
                send_sem=send_sems.at[sem_idx], recv_sem=recv_sems.at[sem_idx],
                device_id=(nbr,), device_id_type=pl.DeviceIdType.MESH,
            )

        def opp_gemm(strip):
            if not do_gemm:
                return
            row = opp_row + strip * m_q
            out_ref[pl.ds(row, m_q), :] = jnp.dot(
                gath_ref[pl.ds(row, m_q), :], w_ref[...],
                preferred_element_type=jnp.float32,
            )

        opp_recv(0, 2, left).wait_recv()
        opp_gemm(0)
        opp_recv(2, 4, right).wait_recv()
        opp_gemm(2)
        opp_recv(1, 3, left).wait_recv()
        opp_gemm(1)
        opp_recv(3, 5, right).wait_recv()
        opp_gemm(3)

        s_r0.wait_send()
        s_l0.wait_send()
        s_r1a.wait_send()
        s_r1b.wait_send()
        s_l1a.wait_send()
        s_l1b.wait_send()

    return pl.pallas_call(
        body,
        out_shape=jax.ShapeDtypeStruct((m_total, n_per), jnp.float32),
        in_specs=[
            pl.BlockSpec(memory_space=pltpu.VMEM),
            pl.BlockSpec(memory_space=pltpu.VMEM),
        ],
        out_specs=pl.BlockSpec(memory_space=pltpu.VMEM),
        scratch_shapes=[
            pltpu.VMEM((m_total, k), jnp.float32),
            pltpu.SemaphoreType.DMA((6,)),
            pltpu.SemaphoreType.DMA((6,)),
        ],
        compiler_params=pltpu.CompilerParams(collective_id=0),
    )(x, w_mat)


# device time: 6340 ns/iter; 8.7101x vs baseline; 8.7101x over previous
import os

import jax
import jax.numpy as jnp
from jax import lax
from jax.experimental import pallas as pl
from jax.experimental.pallas import tpu as pltpu

N_DEV = 4

_KMODE = os.environ.get("KMODE", "full")


def kernel(x, w_mat):
    m_per, k = x.shape
    _, n_per = w_mat.shape
    m_total = N_DEV * m_per
    m_half = m_per // 2
    m_q = m_per // 4
    do_gemm = _KMODE in ("full", "gemm")
    do_comm = _KMODE in ("full", "comm")
    probe = _KMODE if _KMODE in ("bar", "send1", "p0", "bar_any") else None

    if probe == "bar_any":
        def bar_any_body(x_ref, w_ref, out_ref):
            my = lax.axis_index("i")
            left = lax.rem(my - 1 + N_DEV, N_DEV)
            right = lax.rem(my + 1, N_DEV)
            barrier_sem = pltpu.get_barrier_semaphore()
            for nbr in [left, right]:
                pl.semaphore_signal(
                    barrier_sem, inc=1,
                    device_id=(nbr,), device_id_type=pl.DeviceIdType.MESH,
                )
            pl.semaphore_wait(barrier_sem, 2)
        return pl.pallas_call(
            bar_any_body,
            out_shape=jax.ShapeDtypeStruct((m_total, n_per), jnp.float32),
            in_specs=[pl.BlockSpec(memory_space=pl.ANY),
                      pl.BlockSpec(memory_space=pl.ANY)],
            out_specs=pl.BlockSpec(memory_space=pl.ANY),
            compiler_params=pltpu.CompilerParams(collective_id=0),
        )(x, w_mat)

    if probe is not None:
        def probe_body(x_ref, w_ref, out_ref, gath_ref, send_sems, recv_sems):
            my = lax.axis_index("i")
            left = lax.rem(my - 1 + N_DEV, N_DEV)
            right = lax.rem(my + 1, N_DEV)
            my_row = my * m_per
            left_row = left * m_per
            right_row = right * m_per

            barrier_sem = pltpu.get_barrier_semaphore()
            for nbr in [left, right]:
                pl.semaphore_signal(
                    barrier_sem, inc=1,
                    device_id=(nbr,), device_id_type=pl.DeviceIdType.MESH,
                )
            pl.semaphore_wait(barrier_sem, 2)
            if probe == "bar":
                return
            gath_ref[pl.ds(my_row, m_per), :] = x_ref[...]
            s_r0 = pltpu.make_async_remote_copy(
                src_ref=gath_ref.at[pl.ds(my_row, m_per), :],
                dst_ref=gath_ref.at[pl.ds(my_row, m_per), :],
                send_sem=send_sems.at[0], recv_sem=recv_sems.at[0],
                device_id=(right,), device_id_type=pl.DeviceIdType.MESH,
            )
            s_r0.start()
            if probe == "p0":
                s_l0 = pltpu.make_async_remote_copy(
                    src_ref=gath_ref.at[pl.ds(my_row, m_per), :],
                    dst_ref=gath_ref.at[pl.ds(my_row, m_per), :],
                    send_sem=send_sems.at[1], recv_sem=recv_sems.at[1],
                    device_id=(left,), device_id_type=pl.DeviceIdType.MESH,
                )
                s_l0.start()
            r_l0 = pltpu.make_async_remote_copy(
                src_ref=gath_ref.at[pl.ds(left_row, m_per), :],
                dst_ref=gath_ref.at[pl.ds(left_row, m_per), :],
                send_sem=send_sems.at[0], recv_sem=recv_sems.at[0],
                device_id=(left,), device_id_type=pl.DeviceIdType.MESH,
            )
            r_l0.wait_recv()
            if probe == "p0":
                r_r0 = pltpu.make_async_remote_copy(
                    src_ref=gath_ref.at[pl.ds(right_row, m_per), :],
                    dst_ref=gath_ref.at[pl.ds(right_row, m_per), :],
                    send_sem=send_sems.at[1], recv_sem=recv_sems.at[1],
                    device_id=(right,), device_id_type=pl.DeviceIdType.MESH,
                )
                r_r0.wait_recv()
                s_l0.wait_send()
            s_r0.wait_send()

        return pl.pallas_call(
            probe_body,
            out_shape=jax.ShapeDtypeStruct((m_total, n_per), jnp.float32),
            in_specs=[pl.BlockSpec(memory_space=pltpu.VMEM),
                      pl.BlockSpec(memory_space=pltpu.VMEM)],
            out_specs=pl.BlockSpec(memory_space=pltpu.VMEM),
            scratch_shapes=[
                pltpu.VMEM((m_total, k), jnp.float32),
                pltpu.SemaphoreType.DMA((2,)),
                pltpu.SemaphoreType.DMA((2,)),
            ],
            compiler_params=pltpu.CompilerParams(collective_id=0),
        )(x, w_mat)

    if _KMODE == "gemm":
        def gemm_body(x_ref, w_ref, out_ref):
            for c in range(N_DEV):
                out_ref[pl.ds(c * m_per, m_per), :] = jnp.dot(
                    x_ref[...], w_ref[...],
                    preferred_element_type=jnp.float32,
                )
        return pl.pallas_call(
            gemm_body,
            out_shape=jax.ShapeDtypeStruct((m_total, n_per), jnp.float32),
            in_specs=[pl.BlockSpec(memory_space=pltpu.VMEM),
                      pl.BlockSpec(memory_space=pltpu.VMEM)],
            out_specs=pl.BlockSpec(memory_space=pltpu.VMEM),
        )(x, w_mat)

    def body(x_ref, w_ref, out_ref, gath_ref, send_sems, recv_sems):
        my = lax.axis_index("i")
        left = lax.rem(my - 1 + N_DEV, N_DEV)
        right = lax.rem(my + 1, N_DEV)
        opp = lax.rem(my + 2, N_DEV)

        my_row = my * m_per
        left_row = left * m_per
        right_row = right * m_per
        opp_row = opp * m_per

        barrier_sem = pltpu.get_barrier_semaphore()
        for nbr in [left, right]:
            pl.semaphore_signal(
                barrier_sem, inc=1,
                device_id=(nbr,), device_id_type=pl.DeviceIdType.MESH,
            )
        pl.semaphore_wait(barrier_sem, 2)

        gath_ref[pl.ds(my_row, m_per), :] = x_ref[...]

        s_r0 = pltpu.make_async_remote_copy(
            src_ref=gath_ref.at[pl.ds(my_row, m_per), :],
            dst_ref=gath_ref.at[pl.ds(my_row, m_per), :],
            send_sem=send_sems.at[0], recv_sem=recv_sems.at[0],
            device_id=(right,), device_id_type=pl.DeviceIdType.MESH,
        )
        s_r0.start()
        s_l0 = pltpu.make_async_remote_copy(
            src_ref=gath_ref.at[pl.ds(my_row, m_per), :],
            dst_ref=gath_ref.at[pl.ds(my_row, m_per), :],
            send_sem=send_sems.at[1], recv_sem=recv_sems.at[1],
            device_id=(left,), device_id_type=pl.DeviceIdType.MESH,
        )
        s_l0.start()

        if do_gemm:
            out_ref[pl.ds(my_row, m_per), :] = jnp.dot(
                gath_ref[pl.ds(my_row, m_per), :], w_ref[...],
                preferred_element_type=jnp.float32,
            )

        r_l0 = pltpu.make_async_remote_copy(
            src_ref=gath_ref.at[pl.ds(left_row, m_per), :],
            dst_ref=gath_ref.at[pl.ds(left_row, m_per), :],
            send_sem=send_sems.at[0], recv_sem=recv_sems.at[0],
            device_id=(left,), device_id_type=pl.DeviceIdType.MESH,
        )
        r_l0.wait_recv()
        s_r1a = pltpu.make_async_remote_copy(
            src_ref=gath_ref.at[pl.ds(left_row, m_q), :],
            dst_ref=gath_ref.at[pl.ds(left_row, m_q), :],
            send_sem=send_sems.at[2], recv_sem=recv_sems.at[2],
            device_id=(right,), device_id_type=pl.DeviceIdType.MESH,
        )
        s_r1a.start()
        s_r1b = pltpu.make_async_remote_copy(
            src_ref=gath_ref.at[pl.ds(left_row + m_q, m_q), :],
            dst_ref=gath_ref.at[pl.ds(left_row + m_q, m_q), :],
            send_sem=send_sems.at[3], recv_sem=recv_sems.at[3],
            device_id=(right,), device_id_type=pl.DeviceIdType.MESH,
        )
        s_r1b.start()

        r_r0 = pltpu.make_async_remote_copy(
            src_ref=gath_ref.at[pl.ds(right_row, m_per), :],
            dst_ref=gath_ref.at[pl.ds(right_row, m_per), :],
            send_sem=send_sems.at[1], recv_sem=recv_sems.at[1],
            device_id=(right,), device_id_type=pl.DeviceIdType.MESH,
        )
        r_r0.wait_recv()
        s_l1a = pltpu.make_async_remote_copy(
            src_ref=gath_ref.at[pl.ds(right_row + m_half, m_q), :],
            dst_ref=gath_ref.at[pl.ds(right_row + m_half, m_q), :],
            send_sem=send_sems.at[4], recv_sem=recv_sems.at[4],
            device_id=(left,), device_id_type=pl.DeviceIdType.MESH,
        )
        s_l1a.start()
        s_l1b = pltpu.make_async_remote_copy(
            src_ref=gath_ref.at[pl.ds(right_row + m_half + m_q, m_q), :],
            dst_ref=gath_ref.at[pl.ds(right_row + m_half + m_q, m_q), :],
            send_sem=send_sems.at[5], recv_sem=recv_sems.at[5],
            device_id=(left,), device_id_type=pl.DeviceIdType.MESH,
        )
        s_l1b.start()

        if do_gemm:
            out_ref[pl.ds(left_row, m_per), :] = jnp.dot(
                gath_ref[pl.ds(left_row, m_per), :], w_ref[...],
                preferred_element_type=jnp.float32,
            )
            out_ref[pl.ds(right_row, m_per), :] = jnp.dot(
                gath_ref[pl.ds(right_row, m_per), :], w_ref[...],
                preferred_element_type=jnp.float32,
            )

        def opp_recv(strip, sem_idx, nbr):
            return pltpu.make_async_remote_copy(
                src_ref=gath_ref.at[pl.ds(opp_row + strip * m_q, m_q), :],
                dst_ref=gath_ref.at[pl.ds(opp_row + strip * m_q, m_q), :],
                send_sem=send_sems.at[sem_idx], recv_sem=recv_sems.at[sem_idx],
                device_id=(nbr,), device_id_type=pl.DeviceIdType.MESH,
            )

        def opp_gemm(strip):
            if not do_gemm:
                return
            row = opp_row + strip * m_q
            out_ref[pl.ds(row, m_q), :] = jnp.dot(
                gath_ref[pl.ds(row, m_q), :], w_ref[...],
                preferred_element_type=jnp.float32,
            )

        opp_recv(0, 2, left).wait_recv()
        opp_gemm(0)
        opp_recv(2, 4, right).wait_recv()
        opp_gemm(2)
        opp_recv(1, 3, left).wait_recv()
        opp_gemm(1)
        opp_recv(3, 5, right).wait_recv()
        opp_gemm(3)

        s_r0.wait_send()
        s_l0.wait_send()
        s_r1a.wait_send()
        s_r1b.wait_send()
        s_l1a.wait_send()
        s_l1b.wait_send()

    return pl.pallas_call(
        body,
        out_shape=jax.ShapeDtypeStruct((m_total, n_per), jnp.float32),
        in_specs=[
            pl.BlockSpec(memory_space=pltpu.VMEM),
            pl.BlockSpec(memory_space=pltpu.VMEM),
        ],
        out_specs=pl.BlockSpec(memory_space=pltpu.VMEM),
        scratch_shapes=[
            pltpu.VMEM((m_total, k), jnp.float32),
            pltpu.SemaphoreType.DMA((6,)),
            pltpu.SemaphoreType.DMA((6,)),
        ],
        compiler_params=pltpu.CompilerParams(collective_id=0),
    )(x, w_mat)


# device time: 4450 ns/iter; 12.4094x vs baseline; 1.4247x over previous
import os

import jax
import jax.numpy as jnp
from jax import lax
from jax.experimental import pallas as pl
from jax.experimental.pallas import tpu as pltpu

N_DEV = 4

_KMODE = os.environ.get("KMODE", "full")


def kernel(x, w_mat):
    m_per, k = x.shape
    _, n_per = w_mat.shape
    m_total = N_DEV * m_per
    m_half = m_per // 2
    m_q = m_per // 4
    do_gemm = _KMODE in ("full", "gemm")
    do_comm = _KMODE in ("full", "comm")
    probe = _KMODE if _KMODE in ("bar", "send1", "p0", "bar_any") else None

    if probe == "bar_any":
        def bar_any_body(x_ref, w_ref, out_ref):
            my = lax.axis_index("i")
            left = lax.rem(my - 1 + N_DEV, N_DEV)
            right = lax.rem(my + 1, N_DEV)
            if os.environ.get("NOBAR") != "1":
                barrier_sem = pltpu.get_barrier_semaphore()
                for nbr in [left, right]:
                    pl.semaphore_signal(
                        barrier_sem, inc=1,
                        device_id=(nbr,), device_id_type=pl.DeviceIdType.MESH,
                    )
                pl.semaphore_wait(barrier_sem, 2)
        return pl.pallas_call(
            bar_any_body,
            out_shape=jax.ShapeDtypeStruct((m_total, n_per), jnp.float32),
            in_specs=[pl.BlockSpec(memory_space=pl.ANY),
                      pl.BlockSpec(memory_space=pl.ANY)],
            out_specs=pl.BlockSpec(memory_space=pl.ANY),
            **(
                {}
                if os.environ.get("NOBAR") == "1"
                else {"compiler_params": pltpu.CompilerParams(collective_id=0)}
            ),
        )(x, w_mat)

    if probe is not None:
        def probe_body(x_ref, w_ref, out_ref, gath_ref, send_sems, recv_sems):
            my = lax.axis_index("i")
            left = lax.rem(my - 1 + N_DEV, N_DEV)
            right = lax.rem(my + 1, N_DEV)
            my_row = my * m_per
            left_row = left * m_per
            right_row = right * m_per

            barrier_sem = pltpu.get_barrier_semaphore()
            for nbr in [left, right]:
                pl.semaphore_signal(
                    barrier_sem, inc=1,
                    device_id=(nbr,), device_id_type=pl.DeviceIdType.MESH,
                )
            pl.semaphore_wait(barrier_sem, 2)
            if probe == "bar":
                return
            gath_ref[pl.ds(my_row, m_per), :] = x_ref[...]
            s_r0 = pltpu.make_async_remote_copy(
                src_ref=gath_ref.at[pl.ds(my_row, m_per), :],
                dst_ref=gath_ref.at[pl.ds(my_row, m_per), :],
                send_sem=send_sems.at[0], recv_sem=recv_sems.at[0],
                device_id=(right,), device_id_type=pl.DeviceIdType.MESH,
            )
            s_r0.start()
            if probe == "p0":
                s_l0 = pltpu.make_async_remote_copy(
                    src_ref=gath_ref.at[pl.ds(my_row, m_per), :],
                    dst_ref=gath_ref.at[pl.ds(my_row, m_per), :],
                    send_sem=send_sems.at[1], recv_sem=recv_sems.at[1],
                    device_id=(left,), device_id_type=pl.DeviceIdType.MESH,
                )
                s_l0.start()
            r_l0 = pltpu.make_async_remote_copy(
                src_ref=gath_ref.at[pl.ds(left_row, m_per), :],
                dst_ref=gath_ref.at[pl.ds(left_row, m_per), :],
                send_sem=send_sems.at[0], recv_sem=recv_sems.at[0],
                device_id=(left,), device_id_type=pl.DeviceIdType.MESH,
            )
            r_l0.wait_recv()
            if probe == "p0":
                r_r0 = pltpu.make_async_remote_copy(
                    src_ref=gath_ref.at[pl.ds(right_row, m_per), :],
                    dst_ref=gath_ref.at[pl.ds(right_row, m_per), :],
                    send_sem=send_sems.at[1], recv_sem=recv_sems.at[1],
                    device_id=(right,), device_id_type=pl.DeviceIdType.MESH,
                )
                r_r0.wait_recv()
                s_l0.wait_send()
            s_r0.wait_send()

        return pl.pallas_call(
            probe_body,
            out_shape=jax.ShapeDtypeStruct((m_total, n_per), jnp.float32),
            in_specs=[pl.BlockSpec(memory_space=pltpu.VMEM),
                      pl.BlockSpec(memory_space=pltpu.VMEM)],
            out_specs=pl.BlockSpec(memory_space=pltpu.VMEM),
            scratch_shapes=[
                pltpu.VMEM((m_total, k), jnp.float32),
                pltpu.SemaphoreType.DMA((2,)),
                pltpu.SemaphoreType.DMA((2,)),
            ],
            compiler_params=pltpu.CompilerParams(collective_id=0),
        )(x, w_mat)

    if _KMODE == "gemm":
        def gemm_body(x_ref, w_ref, out_ref):
            for c in range(N_DEV):
                out_ref[pl.ds(c * m_per, m_per), :] = jnp.dot(
                    x_ref[...], w_ref[...],
                    preferred_element_type=jnp.float32,
                )
        return pl.pallas_call(
            gemm_body,
            out_shape=jax.ShapeDtypeStruct((m_total, n_per), jnp.float32),
            in_specs=[pl.BlockSpec(memory_space=pltpu.VMEM),
                      pl.BlockSpec(memory_space=pltpu.VMEM)],
            out_specs=pl.BlockSpec(memory_space=pltpu.VMEM),
        )(x, w_mat)

    def body(x_ref, w_ref, out_ref, gath_ref, send_sems, recv_sems):
        my = lax.axis_index("i")
        left = lax.rem(my - 1 + N_DEV, N_DEV)
        right = lax.rem(my + 1, N_DEV)
        opp = lax.rem(my + 2, N_DEV)

        my_row = my * m_per
        left_row = left * m_per
        right_row = right * m_per
        opp_row = opp * m_per

        barrier_sem = pltpu.get_barrier_semaphore()
        for nbr in [left, right]:
            pl.semaphore_signal(
                barrier_sem, inc=1,
                device_id=(nbr,), device_id_type=pl.DeviceIdType.MESH,
            )
        pl.semaphore_wait(barrier_sem, 2)

        gath_ref[pl.ds(my_row, m_per), :] = x_ref[...]

        s_r0 = pltpu.make_async_remote_copy(
            src_ref=gath_ref.at[pl.ds(my_row, m_per), :],
            dst_ref=gath_ref.at[pl.ds(my_row, m_per), :],
            send_sem=send_sems.at[0], recv_sem=recv_sems.at[0],
            device_id=(right,), device_id_type=pl.DeviceIdType.MESH,
        )
        s_r0.start()
        s_l0 = pltpu.make_async_remote_copy(
            src_ref=gath_ref.at[pl.ds(my_row, m_per), :],
            dst_ref=gath_ref.at[pl.ds(my_row, m_per), :],
            send_sem=send_sems.at[1], recv_sem=recv_sems.at[1],
            device_id=(left,), device_id_type=pl.DeviceIdType.MESH,
        )
        s_l0.start()

        if do_gemm:
            out_ref[pl.ds(my_row, m_per), :] = jnp.dot(
                gath_ref[pl.ds(my_row, m_per), :], w_ref[...],
                preferred_element_type=jnp.float32,
            )

        r_l0 = pltpu.make_async_remote_copy(
            src_ref=gath_ref.at[pl.ds(left_row, m_per), :],
            dst_ref=gath_ref.at[pl.ds(left_row, m_per), :],
            send_sem=send_sems.at[0], recv_sem=recv_sems.at[0],
            device_id=(left,), device_id_type=pl.DeviceIdType.MESH,
        )
        r_l0.wait_recv()
        s_r1a = pltpu.make_async_remote_copy(
            src_ref=gath_ref.at[pl.ds(left_row, m_q), :],
            dst_ref=gath_ref.at[pl.ds(left_row, m_q), :],
            send_sem=send_sems.at[2], recv_sem=recv_sems.at[2],
            device_id=(right,), device_id_type=pl.DeviceIdType.MESH,
        )
        s_r1a.start()
        s_r1b = pltpu.make_async_remote_copy(
            src_ref=gath_ref.at[pl.ds(left_row + m_q, m_q), :],
            dst_ref=gath_ref.at[pl.ds(left_row + m_q, m_q), :],
            send_sem=send_sems.at[3], recv_sem=recv_sems.at[3],
            device_id=(right,), device_id_type=pl.DeviceIdType.MESH,
        )
        s_r1b.start()

        r_r0 = pltpu.make_async_remote_copy(
            src_ref=gath_ref.at[pl.ds(right_row, m_per), :],
            dst_ref=gath_ref.at[pl.ds(right_row, m_per), :],
            send_sem=send_sems.at[1], recv_sem=recv_sems.at[1],
            device_id=(right,), device_id_type=pl.DeviceIdType.MESH,
        )
        r_r0.wait_recv()
        s_l1a = pltpu.make_async_remote_copy(
            src_ref=gath_ref.at[pl.ds(right_row + m_half, m_q), :],
            dst_ref=gath_ref.at[pl.ds(right_row + m_half, m_q), :],
            send_sem=send_sems.at[4], recv_sem=recv_sems.at[4],
            device_id=(left,), device_id_type=pl.DeviceIdType.MESH,
        )
        s_l1a.start()
        s_l1b = pltpu.make_async_remote_copy(
            src_ref=gath_ref.at[pl.ds(right_row + m_half + m_q, m_q), :],
            dst_ref=gath_ref.at[pl.ds(right_row + m_half + m_q, m_q), :],
            send_sem=send_sems.at[5], recv_sem=recv_sems.at[5],
            device_id=(left,), device_id_type=pl.DeviceIdType.MESH,
        )
        s_l1b.start()

        if do_gemm:
            out_ref[pl.ds(left_row, m_per), :] = jnp.dot(
                gath_ref[pl.ds(left_row, m_per), :], w_ref[...],
                preferred_element_type=jnp.float32,
            )
            out_ref[pl.ds(right_row, m_per), :] = jnp.dot(
                gath_ref[pl.ds(right_row, m_per), :], w_ref[...],
                preferred_element_type=jnp.float32,
            )

        def opp_recv(strip, sem_idx, nbr):
            return pltpu.make_async_remote_copy(
                src_ref=gath_ref.at[pl.ds(opp_row + strip * m_q, m_q), :],
                dst_ref=gath_ref.at[pl.ds(opp_row + strip * m_q, m_q), :],
                send_sem=send_sems.at[sem_idx], recv_sem=recv_sems.at[sem_idx],
                device_id=(nbr,), device_id_type=pl.DeviceIdType.MESH,
            )

        def opp_gemm(strip):
            if not do_gemm:
                return
            row = opp_row + strip * m_q
            out_ref[pl.ds(row, m_q), :] = jnp.dot(
                gath_ref[pl.ds(row, m_q), :], w_ref[...],
                preferred_element_type=jnp.float32,
            )

        opp_recv(0, 2, left).wait_recv()
        opp_gemm(0)
        opp_recv(2, 4, right).wait_recv()
        opp_gemm(2)
        opp_recv(1, 3, left).wait_recv()
        opp_gemm(1)
        opp_recv(3, 5, right).wait_recv()
        opp_gemm(3)

        s_r0.wait_send()
        s_l0.wait_send()
        s_r1a.wait_send()
        s_r1b.wait_send()
        s_l1a.wait_send()
        s_l1b.wait_send()

    return pl.pallas_call(
        body,
        out_shape=jax.ShapeDtypeStruct((m_total, n_per), jnp.float32),
        in_specs=[
            pl.BlockSpec(memory_space=pltpu.VMEM),
            pl.BlockSpec(memory_space=pltpu.VMEM),
        ],
        out_specs=pl.BlockSpec(memory_space=pltpu.VMEM),
        scratch_shapes=[
            pltpu.VMEM((m_total, k), jnp.float32),
            pltpu.SemaphoreType.DMA((6,)),
            pltpu.SemaphoreType.DMA((6,)),
        ],
        compiler_params=pltpu.CompilerParams(collective_id=0),
    )(x, w_mat)
